# Initial kernel scaffold; baseline (speedup 1.0000x reference)
import jax
import jax.numpy as jnp
from jax.experimental import pallas as pl


def kernel(tokens, embedding, positional_embedding, mapping1, mapping2):
    raise NotImplementedError("write your pallas kernel here")


# speedup vs baseline: 1.1305x; 1.1305x over previous
import functools

import jax
import jax.numpy as jnp
from jax import lax
from jax.experimental import pallas as pl
from jax.experimental.pallas import tpu as pltpu
from jax.experimental.pallas import tpu_sc as plsc

NC, NS = 2, 16
NW = NC * NS
SEQ = 200
EMB = 16


def _body(seq, chunk, nchunk, per_w,
          tok_hbm, map1_hbm, map2_hbm, emb_hbm, pe_hbm, out_hbm,
          tok_v, t1_v, t2_v, rows1_v, rows2_v, out_v, pe_v, sem_m, sem_e):
    wid = lax.axis_index("s") * NC + lax.axis_index("c")
    pltpu.sync_copy(pe_hbm.at[pl.ds(0, seq)], pe_v)

    def chunk_body(g, carry):
        base = wid * per_w + g * chunk
        pltpu.sync_copy(tok_hbm.at[pl.ds(base, chunk)], tok_v)
        c1 = pltpu.async_copy(map1_hbm.at[tok_v], t1_v, sem_m)
        c2 = pltpu.async_copy(map2_hbm.at[tok_v], t2_v, sem_m)
        c1.wait()
        c2.wait()
        d1 = pltpu.async_copy(emb_hbm.at[t1_v], rows1_v, sem_e)
        d2 = pltpu.async_copy(emb_hbm.at[t2_v], rows2_v, sem_e)
        d1.wait()
        d2.wait()

        def seq_body(s, c2_):
            def pos_body(p, c3_):
                i = s * seq + p
                out_v[i, :] = (rows1_v[i, :] + rows2_v[i, :]) * 2.0 + pe_v[p, :]
                return c3_

            return lax.fori_loop(0, seq, pos_body, c2_)

        lax.fori_loop(0, chunk // seq, seq_body, 0)
        pltpu.sync_copy(out_v, out_hbm.at[pl.ds(base, chunk)])
        return carry

    lax.fori_loop(0, nchunk, chunk_body, 0)


def kernel(tokens, embedding, positional_embedding, mapping1, mapping2):
    bsz, seqlen = tokens.shape
    n = bsz * seqlen
    assert seqlen == SEQ and n % NW == 0
    per_w = n // NW
    chunk = 1600
    assert per_w % chunk == 0
    nchunk = per_w // chunk

    tok = tokens.reshape(n).astype(jnp.int32)
    pe = positional_embedding.reshape(-1, EMB)

    mesh = plsc.VectorSubcoreMesh(core_axis_name="c", subcore_axis_name="s")
    body = functools.partial(_body, seqlen, chunk, nchunk, per_w)
    out = pl.kernel(
        body,
        out_type=jax.ShapeDtypeStruct((n, EMB), jnp.float32),
        mesh=mesh,
        compiler_params=pltpu.CompilerParams(use_tc_tiling_on_sc=False),
        scratch_types=[
            pltpu.VMEM((chunk,), jnp.int32),
            pltpu.VMEM((chunk,), jnp.int32),
            pltpu.VMEM((chunk,), jnp.int32),
            pltpu.VMEM((chunk, EMB), jnp.float32),
            pltpu.VMEM((chunk, EMB), jnp.float32),
            pltpu.VMEM((chunk, EMB), jnp.float32),
            pltpu.VMEM((SEQ, EMB), jnp.float32),
            pltpu.SemaphoreType.DMA,
            pltpu.SemaphoreType.DMA,
        ],
    )(tok, mapping1, mapping2, embedding, pe)
    return out.reshape(bsz, seqlen, EMB)
